# baseline (speedup 1.0000x reference)
import functools

import jax
import jax.numpy as jnp
from jax import lax
from jax.experimental import pallas as pl
from jax.experimental.pallas import tpu as pltpu
from jax.experimental.pallas import tpu_sc as plsc

NC = 1
NS = 16
NW = NC * NS

BATCH = 4096
HIST = 50
HIST_PAD = 64
EMBED_DIM = 128

ELEMS_PER_W = BATCH // NW
GROUP = 4
N_GROUPS = ELEMS_PER_W // GROUP
NBUF = 4
DEPTH = 2


def _make_sc_gather():
    mesh = plsc.VectorSubcoreMesh(
        core_axis_name="c", subcore_axis_name="s",
        num_cores=NC, num_subcores=NS)

    @functools.partial(
        pl.kernel,
        out_type=jax.ShapeDtypeStruct((BATCH, HIST, EMBED_DIM), jnp.float32),
        mesh=mesh,
        scratch_types=[
            pltpu.VMEM((ELEMS_PER_W * HIST_PAD,), jnp.int32),
        ] + [pltpu.VMEM((GROUP, HIST, EMBED_DIM), jnp.float32)] * NBUF
          + [pltpu.SemaphoreType.DMA] * (2 * NBUF),
    )
    def sc_gather(idx_hbm, table_hbm, out_hbm, idx_v, *bufs_sems):
        bufs = bufs_sems[:NBUF]
        gsems = bufs_sems[NBUF:2 * NBUF]
        ssems = bufs_sems[2 * NBUF:]
        wid = lax.axis_index("s") * NC + lax.axis_index("c")
        el0 = wid * ELEMS_PER_W
        pltpu.sync_copy(
            idx_hbm.at[pl.ds(el0 * HIST_PAD, ELEMS_PER_W * HIST_PAD)], idx_v)

        def start_gathers(g, b):
            for k in range(GROUP):
                idx_c = idx_v.at[pl.ds((g * GROUP + k) * HIST_PAD, HIST)]
                pltpu.async_copy(
                    table_hbm.at[idx_c], bufs[b].at[k], gsems[b])

        def wait_gathers(g, b):
            for k in range(GROUP):
                pltpu.make_async_copy(
                    table_hbm.at[idx_v.at[pl.ds(0, HIST)]], bufs[b].at[k],
                    gsems[b]).wait()

        def start_store(g, b):
            pltpu.async_copy(
                bufs[b], out_hbm.at[pl.ds(el0 + g * GROUP, GROUP)], ssems[b])

        def wait_store(b):
            pltpu.make_async_copy(
                bufs[b], out_hbm.at[pl.ds(el0, GROUP)], ssems[b]).wait()

        def run(g, b):
            gn = g + DEPTH
            if gn < N_GROUPS:
                bn = gn % NBUF
                if gn >= NBUF:
                    wait_store(bn)
                start_gathers(gn, bn)
            wait_gathers(g, b)
            start_store(g, b)

        for gp in range(DEPTH):
            start_gathers(gp, gp % NBUF)
        for g in range(N_GROUPS):
            run(g, g % NBUF)
        for g in range(N_GROUPS - NBUF, N_GROUPS):
            wait_store(g % NBUF)

    return sc_gather


_sc_gather = _make_sc_gather()


def kernel(indices, weight):
    idx_pad = jnp.pad(indices.astype(jnp.int32),
                      ((0, 0), (0, HIST_PAD - HIST)))
    return _sc_gather(idx_pad.reshape(BATCH * HIST_PAD), weight)


# speedup vs baseline: 1.0863x; 1.0863x over previous
import functools

import jax
import jax.numpy as jnp
from jax import lax
from jax.experimental import pallas as pl
from jax.experimental.pallas import tpu as pltpu
from jax.experimental.pallas import tpu_sc as plsc

NC = 2
NS = 16
NW = NC * NS

BATCH = 4096
HIST = 50
HIST_PAD = 64
EMBED_DIM = 128

ELEMS_PER_W = BATCH // NW
GROUP = 2
N_GROUPS = ELEMS_PER_W // GROUP
NBUF = 8
DEPTH = 4


def _make_sc_gather():
    mesh = plsc.VectorSubcoreMesh(
        core_axis_name="c", subcore_axis_name="s",
        num_cores=NC, num_subcores=NS)

    @functools.partial(
        pl.kernel,
        out_type=jax.ShapeDtypeStruct((BATCH, HIST, EMBED_DIM), jnp.float32),
        mesh=mesh,
        scratch_types=[
            pltpu.VMEM((ELEMS_PER_W * HIST_PAD,), jnp.int32),
        ] + [pltpu.VMEM((GROUP, HIST, EMBED_DIM), jnp.float32)] * NBUF
          + [pltpu.SemaphoreType.DMA] * (2 * NBUF),
    )
    def sc_gather(idx_hbm, table_hbm, out_hbm, idx_v, *bufs_sems):
        bufs = bufs_sems[:NBUF]
        gsems = bufs_sems[NBUF:2 * NBUF]
        ssems = bufs_sems[2 * NBUF:]
        wid = lax.axis_index("s") * NC + lax.axis_index("c")
        el0 = wid * ELEMS_PER_W
        pltpu.sync_copy(
            idx_hbm.at[pl.ds(el0 * HIST_PAD, ELEMS_PER_W * HIST_PAD)], idx_v)

        def start_gathers(g, b):
            for k in range(GROUP):
                idx_c = idx_v.at[pl.ds((g * GROUP + k) * HIST_PAD, HIST)]
                pltpu.async_copy(
                    table_hbm.at[idx_c], bufs[b].at[k], gsems[b])

        def wait_gathers(g, b):
            for k in range(GROUP):
                pltpu.make_async_copy(
                    table_hbm.at[idx_v.at[pl.ds(0, HIST)]], bufs[b].at[k],
                    gsems[b]).wait()

        def start_store(g, b):
            pltpu.async_copy(
                bufs[b], out_hbm.at[pl.ds(el0 + g * GROUP, GROUP)], ssems[b])

        def wait_store(b):
            pltpu.make_async_copy(
                bufs[b], out_hbm.at[pl.ds(el0, GROUP)], ssems[b]).wait()

        def run(g, b):
            gn = g + DEPTH
            if gn < N_GROUPS:
                bn = gn % NBUF
                if gn >= NBUF:
                    wait_store(bn)
                start_gathers(gn, bn)
            wait_gathers(g, b)
            start_store(g, b)

        for gp in range(DEPTH):
            start_gathers(gp, gp % NBUF)
        for g in range(N_GROUPS):
            run(g, g % NBUF)
        for g in range(N_GROUPS - NBUF, N_GROUPS):
            wait_store(g % NBUF)

    return sc_gather


_sc_gather = _make_sc_gather()


def kernel(indices, weight):
    idx_pad = jnp.pad(indices.astype(jnp.int32),
                      ((0, 0), (0, HIST_PAD - HIST)))
    return _sc_gather(idx_pad.reshape(BATCH * HIST_PAD), weight)
